# Initial kernel scaffold; baseline (speedup 1.0000x reference)
import math

import numpy as np
import jax
import jax.numpy as jnp
from jax.experimental import pallas as pl

B, L, H, D = 1, 2048, 12, 64
S = 2048
U = min(5 * math.ceil(math.log(L)), S)
SCALE = (1.0 / 0.24) * math.log(S)
BL = 512


def _build_count_matrix():
    try:
        cpu = jax.local_devices(backend="cpu")[0]
        with jax.default_device(cpu):
            idx = jax.random.randint(jax.random.key(42), (B, H, L, U), 0, S)
            idx_np = np.asarray(idx)
    except RuntimeError:
        idx_np = np.asarray(
            jax.random.randint(jax.random.key(42), (B, H, L, U), 0, S))
    counts = np.zeros((H, L, S), dtype=np.int8)
    hh = np.arange(H)[:, None, None]
    ll = np.arange(L)[None, :, None]
    np.add.at(counts, (hh, ll, idx_np[0]), 1)
    return counts


_COUNTS = _build_count_matrix()


def _score_kernel(q_ref, k_ref, c_ref, o_ref):
    q = q_ref[0]
    qc = q - jnp.mean(q, axis=1, keepdims=True)
    qn = jnp.sqrt(jnp.sum(qc * qc, axis=1, keepdims=True))
    qh = qc / jnp.maximum(qn, 1e-12)
    k = k_ref[0]
    kc = k - jnp.mean(k, axis=1, keepdims=True)
    kn = jnp.sqrt(jnp.sum(kc * kc, axis=1, keepdims=True))
    kh = kc / jnp.maximum(kn, 1e-12)
    p = jax.lax.dot_general(
        qh, kh, (((1,), (1,)), ((), ())),
        preferred_element_type=jnp.float32,
        precision=jax.lax.Precision.HIGHEST)
    w = jnp.abs(p) * c_ref[0].astype(jnp.float32)
    o_ref[...] = jnp.sum(w, axis=1)[None, :]


def _attn_kernel(s_ref, q_ref, k_ref, v_ref, o_ref):
    s = s_ref[...]
    iota = jax.lax.broadcasted_iota(jnp.int32, (1, L), 1)

    def body(i, carry):
        vals, rank = carry
        m = jnp.max(vals)
        idx = jnp.min(jnp.where(vals == m, iota, L))
        hit = iota == idx
        rank = jnp.where(hit, i, rank)
        vals = jnp.where(hit, -1.0, vals)
        return vals, rank

    _, rank = jax.lax.fori_loop(
        0, U, body, (s, jnp.full((1, L), -1, jnp.int32)))
    onehot = (jax.lax.broadcasted_iota(jnp.int32, (U, L), 0) == rank
              ).astype(jnp.float32)

    q = q_ref[0]
    k = k_ref[0]
    v = v_ref[0]
    qr = jax.lax.dot_general(
        onehot, q, (((1,), (0,)), ((), ())),
        preferred_element_type=jnp.float32,
        precision=jax.lax.Precision.HIGHEST)
    qn = jnp.sqrt(jnp.sum(qr * qr, axis=1, keepdims=True))
    qhat = qr / jnp.maximum(qn, 1e-12)
    kn = jnp.sqrt(jnp.sum(k * k, axis=1, keepdims=True))
    khat = k / jnp.maximum(kn, 1e-12)
    sc = jax.lax.dot_general(
        qhat, khat, (((1,), (1,)), ((), ())),
        preferred_element_type=jnp.float32,
        precision=jax.lax.Precision.HIGHEST) * SCALE
    m = jnp.max(sc, axis=1, keepdims=True)
    e = jnp.exp(sc - m)
    a = e / jnp.sum(e, axis=1, keepdims=True)
    vs = jax.lax.dot_general(
        a, v, (((1,), (0,)), ((), ())),
        preferred_element_type=jnp.float32,
        precision=jax.lax.Precision.HIGHEST)
    o_ref[0] = jax.lax.dot_general(
        onehot, vs, (((0,), (0,)), ((), ())),
        preferred_element_type=jnp.float32,
        precision=jax.lax.Precision.HIGHEST)


def kernel(queries, keys, values):
    qh = jnp.transpose(queries, (0, 2, 1, 3))[0]
    kh = jnp.transpose(keys, (0, 2, 1, 3))[0]
    vh = jnp.transpose(values, (0, 2, 1, 3))[0]
    counts = jnp.asarray(_COUNTS)

    scores = pl.pallas_call(
        _score_kernel,
        grid=(H, L // BL),
        in_specs=[
            pl.BlockSpec((1, BL, D), lambda h, i: (h, i, 0)),
            pl.BlockSpec((1, S, D), lambda h, i: (h, 0, 0)),
            pl.BlockSpec((1, BL, S), lambda h, i: (h, i, 0)),
        ],
        out_specs=pl.BlockSpec((1, BL), lambda h, i: (h, i)),
        out_shape=jax.ShapeDtypeStruct((H, L), jnp.float32),
    )(qh, kh, counts)

    out_heads = pl.pallas_call(
        _attn_kernel,
        grid=(H,),
        in_specs=[
            pl.BlockSpec((1, L), lambda h: (h, 0)),
            pl.BlockSpec((1, L, D), lambda h: (h, 0, 0)),
            pl.BlockSpec((1, L, D), lambda h: (h, 0, 0)),
            pl.BlockSpec((1, L, D), lambda h: (h, 0, 0)),
        ],
        out_specs=pl.BlockSpec((1, L, D), lambda h: (h, 0, 0)),
        out_shape=jax.ShapeDtypeStruct((H, L, D), jnp.float32),
    )(scores, qh, kh, vh)

    return jnp.transpose(out_heads, (1, 0, 2))[None]


# speedup vs baseline: 38.2875x; 38.2875x over previous
import math

import numpy as np
import jax
import jax.numpy as jnp
from jax.experimental import pallas as pl

B, L, H, D = 1, 2048, 12, 64
S = 2048
U = min(5 * math.ceil(math.log(L)), S)
SCALE = (1.0 / 0.24) * math.log(S)
BL = 512


def _build_count_matrix():
    try:
        cpu = jax.local_devices(backend="cpu")[0]
        with jax.default_device(cpu):
            idx = jax.random.randint(jax.random.key(42), (B, H, L, U), 0, S)
            idx_np = np.asarray(idx)
    except RuntimeError:
        idx_np = np.asarray(
            jax.random.randint(jax.random.key(42), (B, H, L, U), 0, S))
    counts = np.zeros((H, L, S), dtype=np.int8)
    hh = np.arange(H)[:, None, None]
    ll = np.arange(L)[None, :, None]
    np.add.at(counts, (hh, ll, idx_np[0]), 1)
    return counts


_COUNTS = _build_count_matrix()


def _score_kernel(q_ref, k_ref, c_ref, o_ref):
    q = q_ref[0]
    qc = q - jnp.mean(q, axis=1, keepdims=True)
    qn = jnp.sqrt(jnp.sum(qc * qc, axis=1, keepdims=True))
    qh = qc / jnp.maximum(qn, 1e-12)
    k = k_ref[0]
    kc = k - jnp.mean(k, axis=1, keepdims=True)
    kn = jnp.sqrt(jnp.sum(kc * kc, axis=1, keepdims=True))
    kh = kc / jnp.maximum(kn, 1e-12)
    p = jax.lax.dot_general(
        qh, kh, (((1,), (1,)), ((), ())),
        preferred_element_type=jnp.float32,
        precision=jax.lax.Precision.HIGHEST)
    w = jnp.abs(p) * c_ref[0].astype(jnp.float32)
    o_ref[0] = jnp.sum(w, axis=1)[None, :]


def _attn_kernel(s_ref, q_ref, k_ref, v_ref, o_ref):
    s = s_ref[0]
    iota = jax.lax.broadcasted_iota(jnp.int32, (1, L), 1)

    def body(i, carry):
        vals, rank = carry
        m = jnp.max(vals)
        idx = jnp.min(jnp.where(vals == m, iota, L))
        hit = iota == idx
        rank = jnp.where(hit, i, rank)
        vals = jnp.where(hit, -1.0, vals)
        return vals, rank

    _, rank = jax.lax.fori_loop(
        0, U, body, (s, jnp.full((1, L), -1, jnp.int32)))
    onehot = (jax.lax.broadcasted_iota(jnp.int32, (U, L), 0) == rank
              ).astype(jnp.float32)

    q = q_ref[0]
    k = k_ref[0]
    v = v_ref[0]
    qr = jax.lax.dot_general(
        onehot, q, (((1,), (0,)), ((), ())),
        preferred_element_type=jnp.float32,
        precision=jax.lax.Precision.HIGHEST)
    qn = jnp.sqrt(jnp.sum(qr * qr, axis=1, keepdims=True))
    qhat = qr / jnp.maximum(qn, 1e-12)
    kn = jnp.sqrt(jnp.sum(k * k, axis=1, keepdims=True))
    khat = k / jnp.maximum(kn, 1e-12)
    sc = jax.lax.dot_general(
        qhat, khat, (((1,), (1,)), ((), ())),
        preferred_element_type=jnp.float32,
        precision=jax.lax.Precision.HIGHEST) * SCALE
    m = jnp.max(sc, axis=1, keepdims=True)
    e = jnp.exp(sc - m)
    a = e / jnp.sum(e, axis=1, keepdims=True)
    vs = jax.lax.dot_general(
        a, v, (((1,), (0,)), ((), ())),
        preferred_element_type=jnp.float32,
        precision=jax.lax.Precision.HIGHEST)
    o_ref[0] = jax.lax.dot_general(
        onehot, vs, (((0,), (0,)), ((), ())),
        preferred_element_type=jnp.float32,
        precision=jax.lax.Precision.HIGHEST)


def kernel(queries, keys, values):
    qh = jnp.transpose(queries, (0, 2, 1, 3))[0]
    kh = jnp.transpose(keys, (0, 2, 1, 3))[0]
    vh = jnp.transpose(values, (0, 2, 1, 3))[0]
    counts = jnp.asarray(_COUNTS)

    scores = pl.pallas_call(
        _score_kernel,
        grid=(H, L // BL),
        in_specs=[
            pl.BlockSpec((1, BL, D), lambda h, i: (h, i, 0)),
            pl.BlockSpec((1, S, D), lambda h, i: (h, 0, 0)),
            pl.BlockSpec((1, BL, S), lambda h, i: (h, i, 0)),
        ],
        out_specs=pl.BlockSpec((1, 1, BL), lambda h, i: (h, 0, i)),
        out_shape=jax.ShapeDtypeStruct((H, 1, L), jnp.float32),
    )(qh, kh, counts)

    out_heads = pl.pallas_call(
        _attn_kernel,
        grid=(H,),
        in_specs=[
            pl.BlockSpec((1, 1, L), lambda h: (h, 0, 0)),
            pl.BlockSpec((1, L, D), lambda h: (h, 0, 0)),
            pl.BlockSpec((1, L, D), lambda h: (h, 0, 0)),
            pl.BlockSpec((1, L, D), lambda h: (h, 0, 0)),
        ],
        out_specs=pl.BlockSpec((1, L, D), lambda h: (h, 0, 0)),
        out_shape=jax.ShapeDtypeStruct((H, L, D), jnp.float32),
    )(scores, qh, kh, vh)

    return jnp.transpose(out_heads, (1, 0, 2))[None]


# speedup vs baseline: 51.5258x; 1.3458x over previous
import functools
import math

import numpy as np
import jax
import jax.numpy as jnp
from jax import lax
from jax.experimental import pallas as pl
from jax.experimental.pallas import tpu as pltpu
from jax.experimental.pallas import tpu_sc as plsc

B, L, H, D = 1, 2048, 12, 64
S = 2048
U = min(5 * math.ceil(math.log(L)), S)
PADU = 64
SCALE = (1.0 / 0.24) * math.log(S)
SCHUNK = 512
NW = 32
ROWS_W = (L * H) // NW
BIG = 10 ** 6


def _build_count_matrix_t():
    try:
        cpu = jax.local_devices(backend="cpu")[0]
        with jax.default_device(cpu):
            idx = jax.random.randint(jax.random.key(42), (B, H, L, U), 0, S)
            idx_np = np.asarray(idx)
    except Exception:
        return None
    counts = np.zeros((H, S, L), dtype=np.int8)
    hh = np.repeat(np.arange(H), L * U)
    ll = np.tile(np.repeat(np.arange(L), U), H)
    np.add.at(counts, (hh, idx_np[0].ravel(), ll), 1)
    return counts


_COUNTS_T = _build_count_matrix_t()


def _count_matrix_t():
    if _COUNTS_T is not None:
        return jnp.asarray(_COUNTS_T)
    idx = jax.random.randint(jax.random.key(42), (B, H, L, U), 0, S)[0]
    hh = jnp.arange(H)[:, None, None] * jnp.ones((1, L, U), jnp.int32)
    ll = jnp.arange(L)[None, :, None] * jnp.ones((H, 1, U), jnp.int32)
    return (jnp.zeros((H, S, L), jnp.int8)
            .at[hh.astype(jnp.int32), idx, ll.astype(jnp.int32)].add(1))


def _score_kernel(q_ref, k_ref, c_ref, o_ref, kh1_ref):
    q = q_ref[0]
    k = k_ref[0]
    qc = q - jnp.mean(q, axis=1, keepdims=True)
    qn = jnp.sqrt(jnp.sum(qc * qc, axis=1, keepdims=True))
    qh1 = qc / jnp.maximum(qn, 1e-12)
    kc = k - jnp.mean(k, axis=1, keepdims=True)
    kn = jnp.sqrt(jnp.sum(kc * kc, axis=1, keepdims=True))
    kh1_ref[...] = kc / jnp.maximum(kn, 1e-12)

    def chunk(j, acc):
        kj = kh1_ref[pl.ds(j * SCHUNK, SCHUNK), :]
        pj = lax.dot_general(
            kj, qh1, (((1,), (1,)), ((), ())),
            preferred_element_type=jnp.float32,
            precision=lax.Precision.HIGHEST)
        cj = c_ref[0, pl.ds(j * SCHUNK, SCHUNK), :].astype(jnp.float32)
        return acc + jnp.sum(jnp.abs(pj) * cj, axis=0, keepdims=True)

    o_ref[0] = lax.fori_loop(
        0, S // SCHUNK, chunk, jnp.zeros((1, L), jnp.float32))


def _attn_kernel(mtop_ref, q_ref, k_ref, v_ref, vsel_ref):
    q = q_ref[0]
    k = k_ref[0]
    v = v_ref[0]
    mrow = mtop_ref[0].astype(jnp.float32)
    eye = (lax.broadcasted_iota(jnp.int32, (PADU, PADU), 0) ==
           lax.broadcasted_iota(jnp.int32, (PADU, PADU), 1)).astype(jnp.float32)
    mcol = lax.dot_general(
        eye, mrow, (((1,), (1,)), ((), ())),
        preferred_element_type=jnp.float32,
        precision=lax.Precision.HIGHEST)
    onehot = (lax.broadcasted_iota(jnp.int32, (PADU, L), 1) ==
              mcol.astype(jnp.int32)).astype(jnp.float32)
    qr = lax.dot_general(
        onehot, q, (((1,), (0,)), ((), ())),
        preferred_element_type=jnp.float32,
        precision=lax.Precision.HIGHEST)
    qrn = jnp.sqrt(jnp.sum(qr * qr, axis=1, keepdims=True))
    qhat = qr / jnp.maximum(qrn, 1e-12)
    k2n = jnp.sqrt(jnp.sum(k * k, axis=1, keepdims=True))
    khat = k / jnp.maximum(k2n, 1e-12)
    sc = lax.dot_general(
        qhat, khat, (((1,), (1,)), ((), ())),
        preferred_element_type=jnp.float32,
        precision=lax.Precision.HIGHEST) * SCALE
    m = jnp.max(sc, axis=1, keepdims=True)
    e = jnp.exp(sc - m)
    a = e / jnp.sum(e, axis=1, keepdims=True)
    vsel_ref[0] = lax.dot_general(
        a, v, (((1,), (0,)), ((), ())),
        preferred_element_type=jnp.float32,
        precision=lax.Precision.HIGHEST)


_SC_CACHE = {}


def _sc_kernels():
    if "topk" in _SC_CACHE:
        return _SC_CACHE["topk"], _SC_CACHE["scatter"]
    _sc_mesh = plsc.VectorSubcoreMesh(core_axis_name="c", subcore_axis_name="s")
    _SC_CACHE["topk"] = _make_sc_topk(_sc_mesh)
    _SC_CACHE["scatter"] = _make_sc_scatter(_sc_mesh)
    return _SC_CACHE["topk"], _SC_CACHE["scatter"]


def _make_sc_topk(_sc_mesh):
    return functools.partial(
        pl.kernel,
        mesh=_sc_mesh,
        out_type=jax.ShapeDtypeStruct((H, PADU), jnp.int32),
        scratch_types=[
            pltpu.VMEM((L,), jnp.float32),
            pltpu.VMEM((PADU,), jnp.int32),
            pltpu.VMEM((32,), jnp.float32),
            pltpu.VMEM((32,), jnp.int32),
        ],
    )(_sc_topk_body)


def _sc_topk_body(scores_hbm, mtop_hbm, srow, sel, rbuf, ibuf):
    wid = lax.axis_index("s") * 2 + lax.axis_index("c")

    @pl.when(wid < H)
    def _():
        h = wid
        pltpu.sync_copy(scores_hbm.at[h], srow)
        lanes = lax.broadcasted_iota(jnp.int32, (16,), 0)
        rbuf[pl.ds(16, 16)] = jnp.full((16,), -3.0, jnp.float32)
        ibuf[pl.ds(16, 16)] = jnp.full((16,), BIG, jnp.int32)
        for t in range(PADU // 16):
            sel[pl.ds(t * 16, 16)] = jnp.full((16,), BIG, jnp.int32)

        def ext(e, _):
            def scan(j, carry):
                best, bidx = carry
                v = srow[pl.ds(j * 16, 16)]
                upd = v > best
                best = jnp.where(upd, v, best)
                bidx = jnp.where(upd, j * 16 + lanes, bidx)
                return best, bidx

            best, bidx = lax.fori_loop(
                0, L // 16, scan,
                (jnp.full((16,), -2.0, jnp.float32),
                 jnp.zeros((16,), jnp.int32)), unroll=4)
            mv = best
            for sh in (8, 4, 2, 1):
                rbuf[pl.ds(0, 16)] = mv
                mv = jnp.maximum(mv, rbuf[pl.ds(sh, 16)])
            m = mv[0]
            cand = jnp.where(best == m, bidx, jnp.full((16,), BIG, jnp.int32))
            for sh in (8, 4, 2, 1):
                ibuf[pl.ds(0, 16)] = cand
                cand = jnp.minimum(cand, ibuf[pl.ds(sh, 16)])
            idx = cand[0]
            ib = jnp.bitwise_and(idx, -16)
            hit = lanes == (idx - ib)
            srow[pl.ds(ib, 16)] = jnp.where(
                hit, -1.0, srow[pl.ds(ib, 16)])
            eb = jnp.bitwise_and(e, -16)
            ehit = lanes == (e - eb)
            sel[pl.ds(eb, 16)] = jnp.where(ehit, idx, sel[pl.ds(eb, 16)])
            return 0

        lax.fori_loop(0, U, ext, 0)
        pltpu.sync_copy(sel, mtop_hbm.at[h])


def _make_sc_scatter(_sc_mesh):
    return functools.partial(
        pl.kernel,
        mesh=_sc_mesh,
        out_type=jax.ShapeDtypeStruct((L * H * D,), jnp.float32),
        scratch_types=[
            pltpu.VMEM((ROWS_W * D,), jnp.float32),
            pltpu.VMEM((H * PADU * D,), jnp.float32),
            pltpu.VMEM((H * PADU,), jnp.int32),
        ],
    )(_sc_scatter_body)


def _sc_scatter_body(vsel_hbm, mtop_hbm, out_hbm, slab, vrows, idxs):
    wid = lax.axis_index("s") * 2 + lax.axis_index("c")
    base = wid * ROWS_W
    pltpu.sync_copy(mtop_hbm, idxs)
    pltpu.sync_copy(vsel_hbm, vrows)

    zeros16 = jnp.zeros((16,), jnp.float32)

    def zbody(i, _):
        slab[pl.ds(i * 16, 16)] = zeros16
        return 0

    lax.fori_loop(0, (ROWS_W * D) // 16, zbody, 0)

    for hh in range(H):
        def sbody(i, _):
            m = idxs[pl.ds(hh * PADU + i, 16)][0]
            row = m * H + hh

            @pl.when((row >= base) & (row < base + ROWS_W))
            def _():
                dst = (row - base) * D
                src = (hh * PADU + i) * D
                for c4 in range(D // 16):
                    slab[pl.ds(dst + c4 * 16, 16)] = (
                        vrows[pl.ds(src + c4 * 16, 16)])

            return 0

        lax.fori_loop(0, U, sbody, 0)

    pltpu.sync_copy(slab, out_hbm.at[pl.ds(base * D, ROWS_W * D)])


def kernel(queries, keys, values):
    _sc_topk, _sc_scatter = _sc_kernels()
    counts_t = _count_matrix_t()
    qh = jnp.transpose(queries, (0, 2, 1, 3))[0]
    kh = jnp.transpose(keys, (0, 2, 1, 3))[0]
    vh = jnp.transpose(values, (0, 2, 1, 3))[0]

    scores = pl.pallas_call(
        _score_kernel,
        grid=(H,),
        in_specs=[
            pl.BlockSpec((1, L, D), lambda h: (h, 0, 0)),
            pl.BlockSpec((1, S, D), lambda h: (h, 0, 0)),
            pl.BlockSpec((1, S, L), lambda h: (h, 0, 0)),
        ],
        out_specs=pl.BlockSpec((1, 1, L), lambda h: (h, 0, 0)),
        out_shape=jax.ShapeDtypeStruct((H, 1, L), jnp.float32),
        scratch_shapes=[pltpu.VMEM((S, D), jnp.float32)],
    )(qh, kh, counts_t)

    mtop = _sc_topk(scores.reshape(H, L))

    vsel = pl.pallas_call(
        _attn_kernel,
        grid=(H,),
        in_specs=[
            pl.BlockSpec((1, 1, PADU), lambda h: (h, 0, 0)),
            pl.BlockSpec((1, L, D), lambda h: (h, 0, 0)),
            pl.BlockSpec((1, L, D), lambda h: (h, 0, 0)),
            pl.BlockSpec((1, L, D), lambda h: (h, 0, 0)),
        ],
        out_specs=pl.BlockSpec((1, PADU, D), lambda h: (h, 0, 0)),
        out_shape=jax.ShapeDtypeStruct((H, PADU, D), jnp.float32),
    )(mtop.reshape(H, 1, PADU), qh, kh, vh)

    out_flat = _sc_scatter(vsel.reshape(H * PADU * D), mtop.reshape(H * PADU))
    return out_flat.reshape(B, L, H, D)


# speedup vs baseline: 54.5892x; 1.0595x over previous
import functools
import math

import numpy as np
import jax
import jax.numpy as jnp
from jax import lax
from jax.experimental import pallas as pl
from jax.experimental.pallas import tpu as pltpu
from jax.experimental.pallas import tpu_sc as plsc

B, L, H, D = 1, 2048, 12, 64
S = 2048
U = min(5 * math.ceil(math.log(L)), S)
PADU = 64
SCALE = (1.0 / 0.24) * math.log(S)
SCHUNK = 512
NW = 32
ROWS_W = (L * H) // NW
BIG = 10 ** 6


def _build_count_matrix_t():
    try:
        cpu = jax.local_devices(backend="cpu")[0]
        with jax.default_device(cpu):
            idx = jax.random.randint(jax.random.key(42), (B, H, L, U), 0, S)
            idx_np = np.asarray(idx)
    except Exception:
        return None
    counts = np.zeros((H, S, L), dtype=np.int8)
    hh = np.repeat(np.arange(H), L * U)
    ll = np.tile(np.repeat(np.arange(L), U), H)
    np.add.at(counts, (hh, idx_np[0].ravel(), ll), 1)
    return counts


_COUNTS_T = _build_count_matrix_t()


def _count_matrix_t():
    if _COUNTS_T is not None:
        return jnp.asarray(_COUNTS_T)
    idx = jax.random.randint(jax.random.key(42), (B, H, L, U), 0, S)[0]
    hh = jnp.arange(H)[:, None, None] * jnp.ones((1, L, U), jnp.int32)
    ll = jnp.arange(L)[None, :, None] * jnp.ones((H, 1, U), jnp.int32)
    return (jnp.zeros((H, S, L), jnp.int8)
            .at[hh.astype(jnp.int32), idx, ll.astype(jnp.int32)].add(1))


def _score_kernel(q_ref, k_ref, c_ref, o_ref):
    q = q_ref[0]
    k = k_ref[0]
    qc = q - jnp.mean(q, axis=1, keepdims=True)
    qn = jnp.sqrt(jnp.sum(qc * qc, axis=1, keepdims=True))
    qh1 = qc / jnp.maximum(qn, 1e-12)
    kc = k - jnp.mean(k, axis=1, keepdims=True)
    kn = jnp.sqrt(jnp.sum(kc * kc, axis=1, keepdims=True))
    kh1 = kc / jnp.maximum(kn, 1e-12)

    def dot_chunk(j):
        kj = kh1[j * SCHUNK:(j + 1) * SCHUNK, :]
        return lax.dot_general(
            kj, qh1, (((1,), (1,)), ((), ())),
            preferred_element_type=jnp.float32,
            precision=lax.Precision.HIGHEST)

    def weigh(j, pj):
        cj = c_ref[0, j * SCHUNK:(j + 1) * SCHUNK, :].astype(jnp.float32)
        return jnp.sum(jnp.abs(pj) * cj, axis=0, keepdims=True)

    acc = jnp.zeros((1, L), jnp.float32)
    p_prev = dot_chunk(0)
    for j in range(1, S // SCHUNK):
        p_cur = dot_chunk(j)
        acc = acc + weigh(j - 1, p_prev)
        p_prev = p_cur
    o_ref[0] = acc + weigh(S // SCHUNK - 1, p_prev)


def _attn_kernel(mtop_ref, q_ref, k_ref, v_ref, vsel_ref):
    q = q_ref[0]
    k = k_ref[0]
    v = v_ref[0]
    mrow = mtop_ref[0].astype(jnp.float32)
    eye = (lax.broadcasted_iota(jnp.int32, (PADU, PADU), 0) ==
           lax.broadcasted_iota(jnp.int32, (PADU, PADU), 1)).astype(jnp.float32)
    mcol = lax.dot_general(
        eye, mrow, (((1,), (1,)), ((), ())),
        preferred_element_type=jnp.float32,
        precision=lax.Precision.HIGHEST)
    onehot = (lax.broadcasted_iota(jnp.int32, (PADU, L), 1) ==
              mcol.astype(jnp.int32)).astype(jnp.float32)
    qr = lax.dot_general(
        onehot, q, (((1,), (0,)), ((), ())),
        preferred_element_type=jnp.float32,
        precision=lax.Precision.HIGHEST)
    qrn = jnp.sqrt(jnp.sum(qr * qr, axis=1, keepdims=True))
    qhat = qr / jnp.maximum(qrn, 1e-12)
    k2n = jnp.sqrt(jnp.sum(k * k, axis=1, keepdims=True))
    khat = k / jnp.maximum(k2n, 1e-12)
    sc = lax.dot_general(
        qhat, khat, (((1,), (1,)), ((), ())),
        preferred_element_type=jnp.float32,
        precision=lax.Precision.HIGHEST) * SCALE
    m = jnp.max(sc, axis=1, keepdims=True)
    e = jnp.exp(sc - m)
    a = e / jnp.sum(e, axis=1, keepdims=True)
    vsel_ref[0] = lax.dot_general(
        a, v, (((1,), (0,)), ((), ())),
        preferred_element_type=jnp.float32,
        precision=lax.Precision.HIGHEST)


_SC_CACHE = {}


def _sc_kernels():
    if "topk" in _SC_CACHE:
        return _SC_CACHE["topk"], _SC_CACHE["scatter"]
    _sc_mesh = plsc.VectorSubcoreMesh(core_axis_name="c", subcore_axis_name="s")
    _SC_CACHE["topk"] = _make_sc_topk(_sc_mesh)
    _SC_CACHE["scatter"] = _make_sc_scatter(_sc_mesh)
    return _SC_CACHE["topk"], _SC_CACHE["scatter"]


def _make_sc_topk(_sc_mesh):
    return functools.partial(
        pl.kernel,
        mesh=_sc_mesh,
        out_type=jax.ShapeDtypeStruct((H, PADU), jnp.int32),
        scratch_types=[
            pltpu.VMEM((L,), jnp.float32),
            pltpu.VMEM((PADU,), jnp.int32),
            pltpu.VMEM((32,), jnp.float32),
            pltpu.VMEM((32,), jnp.int32),
        ],
    )(_sc_topk_body)


def _sc_topk_body(scores_hbm, mtop_hbm, srow, sel, rbuf, ibuf):
    wid = lax.axis_index("s") * 2 + lax.axis_index("c")

    @pl.when(wid < H)
    def _():
        h = wid
        pltpu.sync_copy(scores_hbm.at[h], srow)
        lanes = lax.broadcasted_iota(jnp.int32, (16,), 0)
        rbuf[pl.ds(16, 16)] = jnp.full((16,), -3.0, jnp.float32)
        ibuf[pl.ds(16, 16)] = jnp.full((16,), BIG, jnp.int32)
        for t in range(PADU // 16):
            sel[pl.ds(t * 16, 16)] = jnp.full((16,), BIG, jnp.int32)

        def ext(e, _):
            def scan(j, carry):
                best, bidx = carry
                v = srow[pl.ds(j * 16, 16)]
                upd = v > best
                best = jnp.where(upd, v, best)
                bidx = jnp.where(upd, j * 16 + lanes, bidx)
                return best, bidx

            best, bidx = lax.fori_loop(
                0, L // 16, scan,
                (jnp.full((16,), -2.0, jnp.float32),
                 jnp.zeros((16,), jnp.int32)), unroll=4)
            mv = best
            for sh in (8, 4, 2, 1):
                rbuf[pl.ds(0, 16)] = mv
                mv = jnp.maximum(mv, rbuf[pl.ds(sh, 16)])
            m = mv[0]
            cand = jnp.where(best == m, bidx, jnp.full((16,), BIG, jnp.int32))
            for sh in (8, 4, 2, 1):
                ibuf[pl.ds(0, 16)] = cand
                cand = jnp.minimum(cand, ibuf[pl.ds(sh, 16)])
            idx = cand[0]
            ib = jnp.bitwise_and(idx, -16)
            hit = lanes == (idx - ib)
            srow[pl.ds(ib, 16)] = jnp.where(
                hit, -1.0, srow[pl.ds(ib, 16)])
            eb = jnp.bitwise_and(e, -16)
            ehit = lanes == (e - eb)
            sel[pl.ds(eb, 16)] = jnp.where(ehit, idx, sel[pl.ds(eb, 16)])
            return 0

        lax.fori_loop(0, U, ext, 0)
        pltpu.sync_copy(sel, mtop_hbm.at[h])


def _make_sc_scatter(_sc_mesh):
    return functools.partial(
        pl.kernel,
        mesh=_sc_mesh,
        out_type=jax.ShapeDtypeStruct((L * H * D,), jnp.float32),
        scratch_types=[
            pltpu.VMEM((ROWS_W * D,), jnp.float32),
            pltpu.VMEM((H * PADU * D,), jnp.float32),
            pltpu.VMEM((H * PADU,), jnp.int32),
        ],
    )(_sc_scatter_body)


def _sc_scatter_body(vsel_hbm, mtop_hbm, out_hbm, slab, vrows, idxs):
    wid = lax.axis_index("s") * 2 + lax.axis_index("c")
    base = wid * ROWS_W
    pltpu.sync_copy(mtop_hbm, idxs)
    pltpu.sync_copy(vsel_hbm, vrows)

    zeros16 = jnp.zeros((16,), jnp.float32)

    def zbody(i, _):
        slab[pl.ds(i * 16, 16)] = zeros16
        return 0

    lax.fori_loop(0, (ROWS_W * D) // 16, zbody, 0)

    for hh in range(H):
        def sbody(i, _):
            m = idxs[pl.ds(hh * PADU + i, 16)][0]
            row = m * H + hh

            @pl.when((row >= base) & (row < base + ROWS_W))
            def _():
                dst = (row - base) * D
                src = (hh * PADU + i) * D
                for c4 in range(D // 16):
                    slab[pl.ds(dst + c4 * 16, 16)] = (
                        vrows[pl.ds(src + c4 * 16, 16)])

            return 0

        lax.fori_loop(0, U, sbody, 0)

    pltpu.sync_copy(slab, out_hbm.at[pl.ds(base * D, ROWS_W * D)])


def kernel(queries, keys, values):
    _sc_topk, _sc_scatter = _sc_kernels()
    counts_t = _count_matrix_t()
    qh = jnp.transpose(queries, (0, 2, 1, 3))[0]
    kh = jnp.transpose(keys, (0, 2, 1, 3))[0]
    vh = jnp.transpose(values, (0, 2, 1, 3))[0]

    scores = pl.pallas_call(
        _score_kernel,
        grid=(H,),
        in_specs=[
            pl.BlockSpec((1, L, D), lambda h: (h, 0, 0)),
            pl.BlockSpec((1, S, D), lambda h: (h, 0, 0)),
            pl.BlockSpec((1, S, L), lambda h: (h, 0, 0)),
        ],
        out_specs=pl.BlockSpec((1, 1, L), lambda h: (h, 0, 0)),
        out_shape=jax.ShapeDtypeStruct((H, 1, L), jnp.float32),
    )(qh, kh, counts_t)

    mtop = _sc_topk(scores.reshape(H, L))

    vsel = pl.pallas_call(
        _attn_kernel,
        grid=(H,),
        in_specs=[
            pl.BlockSpec((1, 1, PADU), lambda h: (h, 0, 0)),
            pl.BlockSpec((1, L, D), lambda h: (h, 0, 0)),
            pl.BlockSpec((1, L, D), lambda h: (h, 0, 0)),
            pl.BlockSpec((1, L, D), lambda h: (h, 0, 0)),
        ],
        out_specs=pl.BlockSpec((1, PADU, D), lambda h: (h, 0, 0)),
        out_shape=jax.ShapeDtypeStruct((H, PADU, D), jnp.float32),
    )(mtop.reshape(H, 1, PADU), qh, kh, vh)

    out_flat = _sc_scatter(vsel.reshape(H * PADU * D), mtop.reshape(H * PADU))
    return out_flat.reshape(B, L, H, D)


# speedup vs baseline: 55.2073x; 1.0113x over previous
import functools
import math

import numpy as np
import jax
import jax.numpy as jnp
from jax import lax
from jax.experimental import pallas as pl
from jax.experimental.pallas import tpu as pltpu
from jax.experimental.pallas import tpu_sc as plsc

B, L, H, D = 1, 2048, 12, 64
S = 2048
U = min(5 * math.ceil(math.log(L)), S)
PADU = 48
SCALE = (1.0 / 0.24) * math.log(S)
SCHUNK = 512
NW = 32
ROWS_W = (L * H) // NW
BIG = 10 ** 6


def _build_count_matrix_t():
    try:
        cpu = jax.local_devices(backend="cpu")[0]
        with jax.default_device(cpu):
            idx = jax.random.randint(jax.random.key(42), (B, H, L, U), 0, S)
            idx_np = np.asarray(idx)
    except Exception:
        return None
    counts = np.zeros((H, S, L), dtype=np.int8)
    hh = np.repeat(np.arange(H), L * U)
    ll = np.tile(np.repeat(np.arange(L), U), H)
    np.add.at(counts, (hh, idx_np[0].ravel(), ll), 1)
    return counts


_COUNTS_T = _build_count_matrix_t()


def _count_matrix_t():
    if _COUNTS_T is not None:
        return jnp.asarray(_COUNTS_T)
    idx = jax.random.randint(jax.random.key(42), (B, H, L, U), 0, S)[0]
    hh = jnp.arange(H)[:, None, None] * jnp.ones((1, L, U), jnp.int32)
    ll = jnp.arange(L)[None, :, None] * jnp.ones((H, 1, U), jnp.int32)
    return (jnp.zeros((H, S, L), jnp.int8)
            .at[hh.astype(jnp.int32), idx, ll.astype(jnp.int32)].add(1))


def _score_kernel(q_ref, k_ref, c_ref, o_ref):
    q = q_ref[0]
    k = k_ref[0]
    qc = q - jnp.mean(q, axis=1, keepdims=True)
    qn = jnp.sqrt(jnp.sum(qc * qc, axis=1, keepdims=True))
    qh1 = qc / jnp.maximum(qn, 1e-12)
    kc = k - jnp.mean(k, axis=1, keepdims=True)
    kn = jnp.sqrt(jnp.sum(kc * kc, axis=1, keepdims=True))
    kh1 = kc / jnp.maximum(kn, 1e-12)

    def dot_chunk(j):
        kj = kh1[j * SCHUNK:(j + 1) * SCHUNK, :]
        return lax.dot_general(
            kj, qh1, (((1,), (1,)), ((), ())),
            preferred_element_type=jnp.float32,
            precision=lax.Precision.HIGHEST)

    def weigh(j, pj):
        cj = c_ref[0, j * SCHUNK:(j + 1) * SCHUNK, :].astype(jnp.float32)
        return jnp.sum(jnp.abs(pj) * cj, axis=0, keepdims=True)

    acc = jnp.zeros((1, L), jnp.float32)
    p_prev = dot_chunk(0)
    for j in range(1, S // SCHUNK):
        p_cur = dot_chunk(j)
        acc = acc + weigh(j - 1, p_prev)
        p_prev = p_cur
    o_ref[0] = acc + weigh(S // SCHUNK - 1, p_prev)


def _attn_kernel(mtop_ref, q_ref, k_ref, v_ref, vsel_ref):
    q = q_ref[0]
    k = k_ref[0]
    v = v_ref[0]
    mrow = mtop_ref[0].astype(jnp.float32)
    eye = (lax.broadcasted_iota(jnp.int32, (PADU, PADU), 0) ==
           lax.broadcasted_iota(jnp.int32, (PADU, PADU), 1)).astype(jnp.float32)
    mcol = lax.dot_general(
        eye, mrow, (((1,), (1,)), ((), ())),
        preferred_element_type=jnp.float32,
        precision=lax.Precision.HIGHEST)
    onehot = (lax.broadcasted_iota(jnp.int32, (PADU, L), 1) ==
              mcol.astype(jnp.int32)).astype(jnp.float32)
    qr = lax.dot_general(
        onehot, q, (((1,), (0,)), ((), ())),
        preferred_element_type=jnp.float32,
        precision=lax.Precision.HIGHEST)
    qrn = jnp.sqrt(jnp.sum(qr * qr, axis=1, keepdims=True))
    qhat = qr / jnp.maximum(qrn, 1e-12)
    k2n = jnp.sqrt(jnp.sum(k * k, axis=1, keepdims=True))
    khat = k / jnp.maximum(k2n, 1e-12)
    sc = lax.dot_general(
        qhat, khat, (((1,), (1,)), ((), ())),
        preferred_element_type=jnp.float32,
        precision=lax.Precision.HIGHEST) * SCALE
    m = jnp.max(sc, axis=1, keepdims=True)
    e = jnp.exp(sc - m)
    a = e / jnp.sum(e, axis=1, keepdims=True)
    vsel_ref[0] = lax.dot_general(
        a, v, (((1,), (0,)), ((), ())),
        preferred_element_type=jnp.float32,
        precision=lax.Precision.HIGHEST)


_SC_CACHE = {}


def _sc_kernels():
    if "topk" in _SC_CACHE:
        return _SC_CACHE["topk"], _SC_CACHE["scatter"]
    _sc_mesh = plsc.VectorSubcoreMesh(core_axis_name="c", subcore_axis_name="s")
    _SC_CACHE["topk"] = _make_sc_topk(_sc_mesh)
    _SC_CACHE["scatter"] = _make_sc_scatter(_sc_mesh)
    return _SC_CACHE["topk"], _SC_CACHE["scatter"]


def _make_sc_topk(_sc_mesh):
    return functools.partial(
        pl.kernel,
        mesh=_sc_mesh,
        out_type=jax.ShapeDtypeStruct((H, PADU), jnp.int32),
        scratch_types=[
            pltpu.VMEM((L,), jnp.float32),
            pltpu.VMEM((PADU,), jnp.int32),
            pltpu.VMEM((32,), jnp.float32),
            pltpu.VMEM((32,), jnp.int32),
        ],
    )(_sc_topk_body)


def _sc_topk_body(scores_hbm, mtop_hbm, srow, sel, rbuf, ibuf):
    wid = lax.axis_index("s") * 2 + lax.axis_index("c")

    @pl.when(wid < H)
    def _():
        h = wid
        pltpu.sync_copy(scores_hbm.at[h], srow)
        lanes = lax.broadcasted_iota(jnp.int32, (16,), 0)
        rbuf[pl.ds(16, 16)] = jnp.full((16,), -3.0, jnp.float32)
        ibuf[pl.ds(16, 16)] = jnp.full((16,), BIG, jnp.int32)
        for t in range(PADU // 16):
            sel[pl.ds(t * 16, 16)] = jnp.full((16,), BIG, jnp.int32)

        def ext(e, _):
            def scan(j, carry):
                best, bidx = carry
                v = srow[pl.ds(j * 16, 16)]
                upd = v > best
                best = jnp.where(upd, v, best)
                bidx = jnp.where(upd, j * 16 + lanes, bidx)
                return best, bidx

            best, bidx = lax.fori_loop(
                0, L // 16, scan,
                (jnp.full((16,), -2.0, jnp.float32),
                 jnp.zeros((16,), jnp.int32)), unroll=4)
            mv = best
            for sh in (8, 4, 2, 1):
                rbuf[pl.ds(0, 16)] = mv
                mv = jnp.maximum(mv, rbuf[pl.ds(sh, 16)])
            m = mv[0]
            cand = jnp.where(best == m, bidx, jnp.full((16,), BIG, jnp.int32))
            for sh in (8, 4, 2, 1):
                ibuf[pl.ds(0, 16)] = cand
                cand = jnp.minimum(cand, ibuf[pl.ds(sh, 16)])
            idx = cand[0]
            ib = jnp.bitwise_and(idx, -16)
            hit = lanes == (idx - ib)
            srow[pl.ds(ib, 16)] = jnp.where(
                hit, -1.0, srow[pl.ds(ib, 16)])
            eb = jnp.bitwise_and(e, -16)
            ehit = lanes == (e - eb)
            sel[pl.ds(eb, 16)] = jnp.where(ehit, idx, sel[pl.ds(eb, 16)])
            return 0

        lax.fori_loop(0, U, ext, 0)
        pltpu.sync_copy(sel, mtop_hbm.at[h])


def _make_sc_scatter(_sc_mesh):
    return functools.partial(
        pl.kernel,
        mesh=_sc_mesh,
        out_type=jax.ShapeDtypeStruct((L * H * D,), jnp.float32),
        scratch_types=[
            pltpu.VMEM((ROWS_W * D,), jnp.float32),
            pltpu.VMEM((H * PADU * D,), jnp.float32),
            pltpu.VMEM((H * PADU,), jnp.int32),
        ],
    )(_sc_scatter_body)


def _sc_scatter_body(vsel_hbm, mtop_hbm, out_hbm, slab, vrows, idxs):
    wid = lax.axis_index("s") * 2 + lax.axis_index("c")
    base = wid * ROWS_W
    pltpu.sync_copy(mtop_hbm, idxs)
    pltpu.sync_copy(vsel_hbm, vrows)

    zeros16 = jnp.zeros((16,), jnp.float32)

    def zbody(i, _):
        slab[pl.ds(i * 16, 16)] = zeros16
        return 0

    lax.fori_loop(0, (ROWS_W * D) // 16, zbody, 0)

    for hh in range(H):
        def sbody(i, _):
            m = idxs[pl.ds(hh * PADU + i, 16)][0]
            row = m * H + hh

            @pl.when((row >= base) & (row < base + ROWS_W))
            def _():
                dst = (row - base) * D
                src = (hh * PADU + i) * D
                for c4 in range(D // 16):
                    slab[pl.ds(dst + c4 * 16, 16)] = (
                        vrows[pl.ds(src + c4 * 16, 16)])

            return 0

        lax.fori_loop(0, U, sbody, 0)

    pltpu.sync_copy(slab, out_hbm.at[pl.ds(base * D, ROWS_W * D)])


def kernel(queries, keys, values):
    _sc_topk, _sc_scatter = _sc_kernels()
    counts_t = _count_matrix_t()
    qh = jnp.transpose(queries, (0, 2, 1, 3))[0]
    kh = jnp.transpose(keys, (0, 2, 1, 3))[0]
    vh = jnp.transpose(values, (0, 2, 1, 3))[0]

    scores = pl.pallas_call(
        _score_kernel,
        grid=(H,),
        in_specs=[
            pl.BlockSpec((1, L, D), lambda h: (h, 0, 0)),
            pl.BlockSpec((1, S, D), lambda h: (h, 0, 0)),
            pl.BlockSpec((1, S, L), lambda h: (h, 0, 0)),
        ],
        out_specs=pl.BlockSpec((1, 1, L), lambda h: (h, 0, 0)),
        out_shape=jax.ShapeDtypeStruct((H, 1, L), jnp.float32),
    )(qh, kh, counts_t)

    mtop = _sc_topk(scores.reshape(H, L))

    vsel = pl.pallas_call(
        _attn_kernel,
        grid=(H,),
        in_specs=[
            pl.BlockSpec((1, 1, PADU), lambda h: (h, 0, 0)),
            pl.BlockSpec((1, L, D), lambda h: (h, 0, 0)),
            pl.BlockSpec((1, L, D), lambda h: (h, 0, 0)),
            pl.BlockSpec((1, L, D), lambda h: (h, 0, 0)),
        ],
        out_specs=pl.BlockSpec((1, PADU, D), lambda h: (h, 0, 0)),
        out_shape=jax.ShapeDtypeStruct((H, PADU, D), jnp.float32),
    )(mtop.reshape(H, 1, PADU), qh, kh, vh)

    out_flat = _sc_scatter(vsel.reshape(H * PADU * D), mtop.reshape(H * PADU))
    return out_flat.reshape(B, L, H, D)
